# baseline (speedup 1.0000x reference)
import functools

import jax
import jax.numpy as jnp
from jax import lax
from jax.experimental import pallas as pl
from jax.experimental.pallas import tpu as pltpu
from jax.experimental.pallas import tpu_sc as plsc

VOCAB = 1000
D = 64
B = 4096
S = 200
TW = VOCAB * D

_info = plsc.get_sparse_core_info()
NC, NS = _info.num_cores, _info.num_subcores
NW = NC * NS
BPW = B // NW
LPW = BPW * S
NBLK = BPW // 16


def _emb_kernel(idx_hbm, table_hbm, out_hbm, table_v, idx_v, x0, x1, os0, os1):
    wid = lax.axis_index("s") * NC + lax.axis_index("c")
    b0 = wid * BPW
    pltpu.sync_copy(table_hbm, table_v)
    pltpu.sync_copy(idx_hbm.at[pl.ds(wid * LPW, LPW)], idx_v)

    bufs = (x0, x1)
    osems = (os0, os1)
    lane_off = lax.iota(jnp.int32, 16) * S

    def fill(s, p):
        x = bufs[p]
        for j in range(NBLK):
            rvec = plsc.load_gather(idx_v, [lane_off + (j * 16 * S + s)])
            base = rvec * D
            for d in range(D):
                x[d, pl.ds(16 * j, 16)] = plsc.load_gather(table_v, [base + d])

    def start_out(s, p):
        return pltpu.async_copy(
            bufs[p], out_hbm.at[s, :, pl.ds(b0, BPW)], osems[p]
        )

    fill(0, 0)
    start_out(0, 0)
    fill(1, 1)
    start_out(1, 1)

    def outer(t, carry):
        s0 = 2 + 2 * t
        for p in range(2):
            s = s0 + p
            pltpu.make_async_copy(
                bufs[p], out_hbm.at[s - 2, :, pl.ds(b0, BPW)], osems[p]
            ).wait()
            fill(s, p)
            start_out(s, p)
        return carry

    lax.fori_loop(0, (S - 2) // 2, outer, 0)

    for p in range(2):
        pltpu.make_async_copy(
            bufs[p], out_hbm.at[S - 2 + p, :, pl.ds(b0, BPW)], osems[p]
        ).wait()


@jax.jit
def _emb(idx_flat, table_flat):
    mesh = plsc.VectorSubcoreMesh(core_axis_name="c", subcore_axis_name="s")
    run = functools.partial(
        pl.kernel,
        out_type=jax.ShapeDtypeStruct((S, D, B), jnp.float32),
        mesh=mesh,
        scratch_types=[
            pltpu.VMEM((TW,), jnp.float32),
            pltpu.VMEM((LPW,), jnp.int32),
            pltpu.VMEM((D, BPW), jnp.float32),
            pltpu.VMEM((D, BPW), jnp.float32),
            pltpu.SemaphoreType.DMA,
            pltpu.SemaphoreType.DMA,
        ],
        compiler_params=pltpu.CompilerParams(
            use_tc_tiling_on_sc=True, needs_layout_passes=False
        ),
    )(_emb_kernel)
    return run(idx_flat, table_flat)


def kernel(upos_encoded, embedding_weight):
    idx_flat = upos_encoded.reshape(B * S).astype(jnp.int32)
    table_flat = embedding_weight.reshape(TW)
    out_sdb = _emb(idx_flat, table_flat)
    return jnp.transpose(out_sdb, (2, 0, 1))


# speedup vs baseline: 12.3883x; 12.3883x over previous
import functools

import jax
import jax.numpy as jnp
from jax import lax
from jax.experimental import pallas as pl
from jax.experimental.pallas import tpu as pltpu
from jax.experimental.pallas import tpu_sc as plsc

VOCAB = 1000
D = 64
B = 4096
S = 200
TW = VOCAB * D

_info = plsc.get_sparse_core_info()
NC, NS = _info.num_cores, _info.num_subcores
NW = NC * NS
BPW = B // NW
LPW = BPW * S
NBLK = BPW // 16


def _emb_kernel(idx_hbm, table_hbm, out_hbm, table_v, idx_v, x0, x1, os0, os1):
    wid = lax.axis_index("s") * NC + lax.axis_index("c")
    b0 = wid * BPW
    pltpu.sync_copy(table_hbm, table_v)
    pltpu.sync_copy(idx_hbm.at[pl.ds(wid * LPW, LPW)], idx_v)

    bufs = (x0, x1)
    osems = (os0, os1)
    lane_off = lax.iota(jnp.int32, 16) * S

    def fill(s, p):
        x = bufs[p]
        bases = tuple(
            plsc.load_gather(idx_v, [lane_off + (j * 16 * S + s)]) * D
            for j in range(NBLK)
        )

        @functools.partial(plsc.parallel_loop, 0, D, unroll=8, carry=bases)
        def _dloop(d, bs):
            for j in range(NBLK):
                x[d, pl.ds(16 * j, 16)] = plsc.load_gather(table_v, [bs[j] + d])
            return bs

    def start_out(s, p):
        return pltpu.async_copy(
            bufs[p], out_hbm.at[s, :, pl.ds(b0, BPW)], osems[p]
        )

    fill(0, 0)
    start_out(0, 0)
    fill(1, 1)
    start_out(1, 1)

    def outer(t, carry):
        s0 = 2 + 2 * t
        for p in range(2):
            s = s0 + p
            pltpu.make_async_copy(
                bufs[p], out_hbm.at[s - 2, :, pl.ds(b0, BPW)], osems[p]
            ).wait()
            fill(s, p)
            start_out(s, p)
        return carry

    lax.fori_loop(0, (S - 2) // 2, outer, 0)

    for p in range(2):
        pltpu.make_async_copy(
            bufs[p], out_hbm.at[S - 2 + p, :, pl.ds(b0, BPW)], osems[p]
        ).wait()


@jax.jit
def _emb(idx_flat, table_flat):
    mesh = plsc.VectorSubcoreMesh(core_axis_name="c", subcore_axis_name="s")
    run = functools.partial(
        pl.kernel,
        out_type=jax.ShapeDtypeStruct((S, D, B), jnp.float32),
        mesh=mesh,
        scratch_types=[
            pltpu.VMEM((TW,), jnp.float32),
            pltpu.VMEM((LPW,), jnp.int32),
            pltpu.VMEM((D, BPW), jnp.float32),
            pltpu.VMEM((D, BPW), jnp.float32),
            pltpu.SemaphoreType.DMA,
            pltpu.SemaphoreType.DMA,
        ],
        compiler_params=pltpu.CompilerParams(
            use_tc_tiling_on_sc=True, needs_layout_passes=False
        ),
    )(_emb_kernel)
    return run(idx_flat, table_flat)


def kernel(upos_encoded, embedding_weight):
    idx_flat = upos_encoded.reshape(B * S).astype(jnp.int32)
    table_flat = embedding_weight.reshape(TW)
    out_sdb = _emb(idx_flat, table_flat)
    return jnp.transpose(out_sdb, (2, 0, 1))
